# baseline (speedup 1.0000x reference)
import functools

import jax
import jax.numpy as jnp
from jax import lax
from jax.experimental import pallas as pl
from jax.experimental.pallas import tpu as pltpu
from jax.experimental.pallas import tpu_sc as plsc

_N_LEVELS = 16
_N_FEAT = 2
_OFFSETS = [0, 4913, 14174, 31750, 67687, 136608, 269259, 543884, 1068172,
            1592460, 2116748, 2641036, 3165324, 3689612, 4213900, 4738188,
            5262476]
_RES = [16, 20, 25, 32, 40, 50, 64, 80, 101, 128, 161, 203, 256, 322, 406, 512]
_P2 = 2654435761
_P3 = 805459861

_B = 131072
_NC, _NS, _L = 2, 16, 16
_NW = _NC * _NS
_BPW = _B // _NW
_C = 512
_CHUNKS = _BPW // _C
_G = _C // _L
_OUT_D = _N_LEVELS * _N_FEAT

_N_CACHED = 2
_TAB_ROWS = _OFFSETS[_N_CACHED]
_TAB_ELEMS = ((_TAB_ROWS * _N_FEAT + 7) // 8) * 8

_mesh = plsc.VectorSubcoreMesh(core_axis_name="c", subcore_axis_name="s")


@functools.partial(
    pl.kernel,
    mesh=_mesh,
    out_type=jax.ShapeDtypeStruct((_B * _OUT_D,), jnp.float32),
    scratch_types=[
        pltpu.VMEM((3, _L), jnp.float32),
        pltpu.VMEM((3, _L), jnp.float32),
        pltpu.VMEM((3 * _C,), jnp.float32),
        pltpu.VMEM((_TAB_ELEMS,), jnp.float32),
        [[pltpu.VMEM((_C,), jnp.int32) for _ in range(16)]
         for _ in range(2)],
        pltpu.VMEM((8, _C), jnp.int32),
        pltpu.VMEM((2, 8, _C), jnp.float32),
        pltpu.VMEM((8, _C), jnp.float32),
        [[pltpu.VMEM((_C,), jnp.float32) for _ in range(16)]
         for _ in range(2)],
        pltpu.VMEM((_C * _OUT_D,), jnp.float32),
        pltpu.SemaphoreType.DMA,
        pltpu.SemaphoreType.DMA,
    ],
    compiler_params=pltpu.CompilerParams(needs_layout_passes=False),
)
def _encode_sc(xt_hbm, emb_hbm, mn_hbm, inv_hbm, out_hbm,
               mn_v, inv_v, xyz_v, tab_v, idx_b, idxc_v, w_v, wc_v, rows_b,
               out_v, sem0, sem1):
    wid = lax.axis_index("s") * _NC + lax.axis_index("c")
    base = wid * _BPW
    pltpu.sync_copy(mn_hbm, mn_v)
    pltpu.sync_copy(inv_hbm, inv_v)
    pltpu.sync_copy(emb_hbm.at[pl.ds(0, _TAB_ELEMS)], tab_v)
    iota = lax.iota(jnp.int32, _L)
    iota32 = iota * _OUT_D
    sems = (sem0, sem1)

    def phase_a(l, buf, cached):
        res = _RES[l]
        size = _OFFSETS[l + 1] - _OFFSETS[l]
        off = _OFFSETS[l]
        dense = (res + 1) ** 3 <= size
        r1 = res + 1

        def grp_a(g, cc):
            o = g * _L
            ps, fs = [], []
            for a in range(3):
                xa = xyz_v[pl.ds(a * _C + o, _L)]
                xn = jnp.clip((xa - mn_v[a]) * inv_v[a], 0.0, 1.0)
                scl = xn * jnp.float32(res)
                p = jnp.minimum(scl.astype(jnp.int32), res - 1)
                ps.append(p)
                fs.append(scl - p.astype(jnp.float32))
            px, py, pz = ps
            fx, fy, fz = fs
            wx0 = 1.0 - fx
            wy0 = 1.0 - fy
            wz0 = 1.0 - fz
            wxy = (wx0 * wy0, fx * wy0, wx0 * fy, fx * fy)
            if not dense:
                hx0 = px.astype(jnp.uint32)
                hx1 = hx0 + jnp.uint32(1)
                hy0 = py.astype(jnp.uint32) * jnp.uint32(_P2)
                hy1 = hy0 + jnp.uint32(_P2)
                hz0 = pz.astype(jnp.uint32) * jnp.uint32(_P3)
                hz1 = hz0 + jnp.uint32(_P3)
                msk = jnp.uint32(size - 1)
            else:
                bidx = px + py * r1 + pz * (r1 * r1) + off
            for corner in range(8):
                dx, dy, dz = corner & 1, (corner >> 1) & 1, (corner >> 2) & 1
                if dense:
                    idx = bidx + (dx + dy * r1 + dz * r1 * r1)
                else:
                    h = ((hx1 if dx else hx0) ^ (hy1 if dy else hy0)
                         ^ (hz1 if dz else hz0))
                    idx = ((h & msk) + jnp.uint32(off)).astype(jnp.int32)
                w = wxy[dy * 2 + dx] * (fz if dz else wz0)
                e = idx + idx
                if cached:
                    idxc_v[corner, pl.ds(o, _L)] = e
                    wc_v[corner, pl.ds(o, _L)] = w
                else:
                    idx_b[buf][2 * corner][pl.ds(o, _L)] = e
                    idx_b[buf][2 * corner + 1][pl.ds(o, _L)] = e + 1
                    w_v[buf, corner, pl.ds(o, _L)] = w
            return cc

        lax.fori_loop(0, _G, grp_a, 0)

    def fire(buf):
        return [pltpu.async_copy(emb_hbm.at[idx_b[buf][k]],
                                 rows_b[buf][k], sems[buf])
                for k in range(16)]

    def phase_c_dma(l, buf):
        def grp_c(g, cc):
            o = g * _L
            acc0 = jnp.zeros((_L,), jnp.float32)
            acc1 = jnp.zeros((_L,), jnp.float32)
            for corner in range(8):
                w = w_v[buf, corner, pl.ds(o, _L)]
                v0 = rows_b[buf][2 * corner][pl.ds(o, _L)]
                v1 = rows_b[buf][2 * corner + 1][pl.ds(o, _L)]
                acc0 = acc0 + w * v0
                acc1 = acc1 + w * v1
            ovec = iota32 + (o * _OUT_D + 2 * l)
            plsc.store_scatter(out_v, [ovec], acc0)
            plsc.store_scatter(out_v, [ovec + 1], acc1)
            return cc

        lax.fori_loop(0, _G, grp_c, 0)

    def phase_c_cached(l):
        def grp_c(g, cc):
            o = g * _L
            acc0 = jnp.zeros((_L,), jnp.float32)
            acc1 = jnp.zeros((_L,), jnp.float32)
            for corner in range(8):
                w = wc_v[corner, pl.ds(o, _L)]
                evec = idxc_v[corner, pl.ds(o, _L)]
                v0 = plsc.load_gather(tab_v, [evec])
                v1 = plsc.load_gather(tab_v, [evec + 1])
                acc0 = acc0 + w * v0
                acc1 = acc1 + w * v1
            ovec = iota32 + (o * _OUT_D + 2 * l)
            plsc.store_scatter(out_v, [ovec], acc0)
            plsc.store_scatter(out_v, [ovec + 1], acc1)
            return cc

        lax.fori_loop(0, _G, grp_c, 0)

    def chunk_body(ci, carry):
        cbase = base + ci * _C
        for a in range(3):
            pltpu.sync_copy(xt_hbm.at[pl.ds(a * _B + cbase, _C)],
                            xyz_v.at[pl.ds(a * _C, _C)])
        pend = None
        ndma = 0
        order = [2, 0, 3, 1] + list(range(4, _N_LEVELS))
        for l in order:
            cached = l < _N_CACHED
            buf = ndma % 2
            phase_a(l, buf, cached)
            if cached:
                phase_c_cached(l)
            else:
                copies = fire(buf)
                ndma += 1
                if pend is not None:
                    pl_, pb_, pc_ = pend
                    for cp in pc_:
                        cp.wait()
                    phase_c_dma(pl_, pb_)
                pend = (l, buf, copies)
        pl_, pb_, pc_ = pend
        for cp in pc_:
            cp.wait()
        phase_c_dma(pl_, pb_)
        pltpu.sync_copy(out_v, out_hbm.at[pl.ds(cbase * _OUT_D, _C * _OUT_D)])
        return carry

    lax.fori_loop(0, _CHUNKS, chunk_body, 0)


def kernel(xyz, embeddings, min_xyz, max_xyz):
    xt = jnp.transpose(xyz).reshape(-1)
    embf = embeddings.reshape(-1)
    inv = 1.0 / (max_xyz - min_xyz)
    mn3 = jnp.broadcast_to(min_xyz[:, None], (3, _L))
    inv3 = jnp.broadcast_to(inv[:, None], (3, _L))
    return _encode_sc(xt, embf, mn3, inv3).reshape(_B, _OUT_D)


# speedup vs baseline: 4.2097x; 4.2097x over previous
import functools

import jax
import jax.numpy as jnp
from jax import lax
from jax.experimental import pallas as pl
from jax.experimental.pallas import tpu as pltpu
from jax.experimental.pallas import tpu_sc as plsc

_N_LEVELS = 16
_N_FEAT = 2
_OFFSETS = [0, 4913, 14174, 31750, 67687, 136608, 269259, 543884, 1068172,
            1592460, 2116748, 2641036, 3165324, 3689612, 4213900, 4738188,
            5262476]
_RES = [16, 20, 25, 32, 40, 50, 64, 80, 101, 128, 161, 203, 256, 322, 406, 512]
_P2 = 2654435761
_P3 = 805459861

_B = 131072
_NC, _NS, _L = 2, 16, 16
_NW = _NC * _NS
_BPW = _B // _NW
_C = 512
_CHUNKS = _BPW // _C
_G = _C // _L
_OUT_D = _N_LEVELS * _N_FEAT

_N_CACHED = 2
_TAB_ROWS = _OFFSETS[_N_CACHED]
_TAB_PAD = ((_TAB_ROWS + 7) // 8) * 8
_TAB_ELEMS = _TAB_PAD * 2
_VP_BLOCKS = (_OFFSETS[-1] + 127) // 128
_VP = _VP_BLOCKS * 128
_OUT_RB = _OUT_D // 8
_PB = _B // 128

_mesh = plsc.VectorSubcoreMesh(core_axis_name="c", subcore_axis_name="s")


@functools.partial(
    pl.kernel,
    mesh=_mesh,
    out_type=jax.ShapeDtypeStruct((_B * _OUT_D,), jnp.float32),
    scratch_types=[
        pltpu.VMEM((3, _L), jnp.float32),
        pltpu.VMEM((3, _L), jnp.float32),
        pltpu.VMEM((3 * _C,), jnp.float32),
        pltpu.VMEM((_TAB_ELEMS,), jnp.float32),
        [[pltpu.VMEM((_C,), jnp.int32) for _ in range(16)]
         for _ in range(2)],
        pltpu.VMEM((8, _C), jnp.int32),
        pltpu.VMEM((2, 8, _C), jnp.float32),
        pltpu.VMEM((8, _C), jnp.float32),
        [[pltpu.VMEM((_C,), jnp.float32) for _ in range(16)]
         for _ in range(2)],
        pltpu.VMEM((_C * _OUT_D,), jnp.float32),
        pltpu.SemaphoreType.DMA,
        pltpu.SemaphoreType.DMA,
    ],
    compiler_params=pltpu.CompilerParams(needs_layout_passes=False),
)
def _encode_sc(xt_hbm, emb_hbm, mn_hbm, inv_hbm, out_hbm,
               mn_v, inv_v, xyz_v, tab_v, idx_b, idxc_v, w_v, wc_v, rows_b,
               out_v, sem0, sem1):
    wid = lax.axis_index("s") * _NC + lax.axis_index("c")
    base = wid * _BPW
    pltpu.sync_copy(mn_hbm, mn_v)
    pltpu.sync_copy(inv_hbm, inv_v)
    pltpu.sync_copy(emb_hbm.at[pl.ds(0, _TAB_PAD)], tab_v.at[pl.ds(0, _TAB_PAD)])
    pltpu.sync_copy(emb_hbm.at[pl.ds(_VP, _TAB_PAD)],
                    tab_v.at[pl.ds(_TAB_PAD, _TAB_PAD)])
    iota = lax.iota(jnp.int32, _L)
    sems = (sem0, sem1)

    def phase_a(l, buf, cached):
        res = _RES[l]
        size = _OFFSETS[l + 1] - _OFFSETS[l]
        off = _OFFSETS[l]
        dense = (res + 1) ** 3 <= size
        r1 = res + 1

        def grp_a(g, cc):
            o = g * _L
            ps, fs = [], []
            for a in range(3):
                xa = xyz_v[pl.ds(a * _C + o, _L)]
                xn = jnp.clip((xa - mn_v[a]) * inv_v[a], 0.0, 1.0)
                scl = xn * jnp.float32(res)
                p = jnp.minimum(scl.astype(jnp.int32), res - 1)
                ps.append(p)
                fs.append(scl - p.astype(jnp.float32))
            px, py, pz = ps
            fx, fy, fz = fs
            wx0 = 1.0 - fx
            wy0 = 1.0 - fy
            wz0 = 1.0 - fz
            wxy = (wx0 * wy0, fx * wy0, wx0 * fy, fx * fy)
            if not dense:
                hx0 = px.astype(jnp.uint32)
                hx1 = hx0 + jnp.uint32(1)
                hy0 = py.astype(jnp.uint32) * jnp.uint32(_P2)
                hy1 = hy0 + jnp.uint32(_P2)
                hz0 = pz.astype(jnp.uint32) * jnp.uint32(_P3)
                hz1 = hz0 + jnp.uint32(_P3)
                msk = jnp.uint32(size - 1)
            else:
                bidx = px + py * r1 + pz * (r1 * r1) + off
            for corner in range(8):
                dx, dy, dz = corner & 1, (corner >> 1) & 1, (corner >> 2) & 1
                if dense:
                    idx = bidx + (dx + dy * r1 + dz * r1 * r1)
                else:
                    h = ((hx1 if dx else hx0) ^ (hy1 if dy else hy0)
                         ^ (hz1 if dz else hz0))
                    idx = ((h & msk) + jnp.uint32(off)).astype(jnp.int32)
                w = wxy[dy * 2 + dx] * (fz if dz else wz0)
                e = idx
                if cached:
                    idxc_v[corner, pl.ds(o, _L)] = e
                    wc_v[corner, pl.ds(o, _L)] = w
                else:
                    idx_b[buf][2 * corner][pl.ds(o, _L)] = e
                    idx_b[buf][2 * corner + 1][pl.ds(o, _L)] = e + _VP
                    w_v[buf, corner, pl.ds(o, _L)] = w
            return cc

        lax.fori_loop(0, _G, grp_a, 0)

    def fire(buf):
        return [pltpu.async_copy(emb_hbm.at[idx_b[buf][k]],
                                 rows_b[buf][k], sems[buf])
                for k in range(16)]

    def phase_c_dma(l, buf):
        def grp_c(g, cc):
            o = g * _L
            acc0 = jnp.zeros((_L,), jnp.float32)
            acc1 = jnp.zeros((_L,), jnp.float32)
            for corner in range(8):
                w = w_v[buf, corner, pl.ds(o, _L)]
                v0 = rows_b[buf][2 * corner][pl.ds(o, _L)]
                v1 = rows_b[buf][2 * corner + 1][pl.ds(o, _L)]
                acc0 = acc0 + w * v0
                acc1 = acc1 + w * v1
            r = l // 4
            fi = (2 * l) & 7
            ovec = (((r * (_C // 128) + g // 8) * 8 + fi) * 128
                    + (o & 127)) + iota
            plsc.store_scatter(out_v, [ovec], acc0)
            plsc.store_scatter(out_v, [ovec + 128], acc1)
            return cc

        lax.fori_loop(0, _G, grp_c, 0)

    def phase_c_cached(l):
        def grp_c(g, cc):
            o = g * _L
            acc0 = jnp.zeros((_L,), jnp.float32)
            acc1 = jnp.zeros((_L,), jnp.float32)
            for corner in range(8):
                w = wc_v[corner, pl.ds(o, _L)]
                evec = idxc_v[corner, pl.ds(o, _L)]
                v0 = plsc.load_gather(tab_v, [evec])
                v1 = plsc.load_gather(tab_v, [evec + _TAB_PAD])
                acc0 = acc0 + w * v0
                acc1 = acc1 + w * v1
            r = l // 4
            fi = (2 * l) & 7
            ovec = (((r * (_C // 128) + g // 8) * 8 + fi) * 128
                    + (o & 127)) + iota
            plsc.store_scatter(out_v, [ovec], acc0)
            plsc.store_scatter(out_v, [ovec + 128], acc1)
            return cc

        lax.fori_loop(0, _G, grp_c, 0)

    def chunk_body(ci, carry):
        cbase = base + ci * _C
        for a in range(3):
            pltpu.sync_copy(xt_hbm.at[pl.ds(a * _B + cbase, _C)],
                            xyz_v.at[pl.ds(a * _C, _C)])
        pend = None
        ndma = 0
        order = [2, 0, 3, 1] + list(range(4, _N_LEVELS))
        for l in order:
            cached = l < _N_CACHED
            buf = ndma % 2
            phase_a(l, buf, cached)
            if cached:
                phase_c_cached(l)
            else:
                copies = fire(buf)
                ndma += 1
                if pend is not None:
                    pl_, pb_, pc_ = pend
                    for cp in pc_:
                        cp.wait()
                    phase_c_dma(pl_, pb_)
                pend = (l, buf, copies)
        pl_, pb_, pc_ = pend
        for cp in pc_:
            cp.wait()
        phase_c_dma(pl_, pb_)
        for r in range(_OUT_RB):
            pltpu.sync_copy(
                out_v.at[pl.ds(r * (_C // 128) * 1024, (_C // 128) * 1024)],
                out_hbm.at[pl.ds((r * _PB + cbase // 128) * 1024,
                                 (_C // 128) * 1024)])
        return carry

    lax.fori_loop(0, _CHUNKS, chunk_body, 0)


def kernel(xyz, embeddings, min_xyz, max_xyz):
    xt = jnp.transpose(xyz).reshape(-1)
    embp = jnp.pad(embeddings, ((0, _VP - _OFFSETS[-1]), (0, 0)))
    embp = embp.T.reshape(-1)
    inv = 1.0 / (max_xyz - min_xyz)
    mn3 = jnp.broadcast_to(min_xyz[:, None], (3, _L))
    inv3 = jnp.broadcast_to(inv[:, None], (3, _L))
    flat = _encode_sc(xt, embp, mn3, inv3)
    return (flat.reshape(_OUT_RB, _PB, 8, 128)
            .transpose(1, 3, 0, 2).reshape(_B, _OUT_D))


# speedup vs baseline: 5.9471x; 1.4127x over previous
import functools

import jax
import jax.numpy as jnp
from jax import lax
from jax.experimental import pallas as pl
from jax.experimental.pallas import tpu as pltpu
from jax.experimental.pallas import tpu_sc as plsc

_N_LEVELS = 16
_N_FEAT = 2
_OFFSETS = [0, 4913, 14174, 31750, 67687, 136608, 269259, 543884, 1068172,
            1592460, 2116748, 2641036, 3165324, 3689612, 4213900, 4738188,
            5262476]
_RES = [16, 20, 25, 32, 40, 50, 64, 80, 101, 128, 161, 203, 256, 322, 406, 512]
_P2 = 2654435761
_P3 = 805459861

_B = 131072
_NC, _NS, _L = 2, 16, 16
_NW = _NC * _NS
_BPW = _B // _NW
_C = 512
_CHUNKS = _BPW // _C
_G = _C // _L
_OUT_D = _N_LEVELS * _N_FEAT

_N_CACHED = 2
_TAB_ROWS = _OFFSETS[_N_CACHED]
_TAB_PAD = ((_TAB_ROWS + 7) // 8) * 8
_TAB_ELEMS = _TAB_PAD * 2
_VP_BLOCKS = (_OFFSETS[-1] + 127) // 128
_VP = _VP_BLOCKS * 128
_OUT_RB = _OUT_D // 8
_PB = _B // 128

_mesh = plsc.VectorSubcoreMesh(core_axis_name="c", subcore_axis_name="s")


@functools.partial(
    pl.kernel,
    mesh=_mesh,
    out_type=jax.ShapeDtypeStruct((_B * _OUT_D,), jnp.float32),
    scratch_types=[
        pltpu.VMEM((3, _L), jnp.float32),
        pltpu.VMEM((3, _L), jnp.float32),
        pltpu.VMEM((3 * _C,), jnp.float32),
        pltpu.VMEM((_TAB_ELEMS,), jnp.float32),
        [[pltpu.VMEM((_C,), jnp.int32) for _ in range(8)]
         for _ in range(2)],
        pltpu.VMEM((8, _C), jnp.int32),
        pltpu.VMEM((2, 8, _C), jnp.float32),
        pltpu.VMEM((8, _C), jnp.float32),
        [[pltpu.VMEM((_C,), jnp.float32) for _ in range(16)]
         for _ in range(2)],
        pltpu.VMEM((_C * _OUT_D,), jnp.float32),
        pltpu.SemaphoreType.DMA,
        pltpu.SemaphoreType.DMA,
    ],
    compiler_params=pltpu.CompilerParams(needs_layout_passes=False),
)
def _encode_sc(xt_hbm, emb0_hbm, emb1_hbm, mn_hbm, inv_hbm, out_hbm,
               mn_v, inv_v, xyz_v, tab_v, idx_b, idxc_v, w_v, wc_v, rows_b,
               out_v, sem0, sem1):
    wid = lax.axis_index("s") * _NC + lax.axis_index("c")
    base = wid * _BPW
    pltpu.sync_copy(mn_hbm, mn_v)
    pltpu.sync_copy(inv_hbm, inv_v)
    pltpu.sync_copy(emb0_hbm.at[pl.ds(0, _TAB_PAD)],
                    tab_v.at[pl.ds(0, _TAB_PAD)])
    pltpu.sync_copy(emb1_hbm.at[pl.ds(0, _TAB_PAD)],
                    tab_v.at[pl.ds(_TAB_PAD, _TAB_PAD)])
    iota = lax.iota(jnp.int32, _L)
    sems = (sem0, sem1)

    def phase_a(l, buf, cached):
        res = _RES[l]
        size = _OFFSETS[l + 1] - _OFFSETS[l]
        off = _OFFSETS[l]
        dense = (res + 1) ** 3 <= size
        r1 = res + 1

        def grp_a(g, cc):
            o = g * _L
            ps, fs = [], []
            for a in range(3):
                xa = xyz_v[pl.ds(a * _C + o, _L)]
                xn = jnp.clip((xa - mn_v[a]) * inv_v[a], 0.0, 1.0)
                scl = xn * jnp.float32(res)
                p = jnp.minimum(scl.astype(jnp.int32), res - 1)
                ps.append(p)
                fs.append(scl - p.astype(jnp.float32))
            px, py, pz = ps
            fx, fy, fz = fs
            wx0 = 1.0 - fx
            wy0 = 1.0 - fy
            wz0 = 1.0 - fz
            wxy = (wx0 * wy0, fx * wy0, wx0 * fy, fx * fy)
            if not dense:
                hx0 = px.astype(jnp.uint32)
                hx1 = hx0 + jnp.uint32(1)
                hy0 = py.astype(jnp.uint32) * jnp.uint32(_P2)
                hy1 = hy0 + jnp.uint32(_P2)
                hz0 = pz.astype(jnp.uint32) * jnp.uint32(_P3)
                hz1 = hz0 + jnp.uint32(_P3)
                msk = jnp.uint32(size - 1)
            else:
                bidx = px + py * r1 + pz * (r1 * r1) + off
            for corner in range(8):
                dx, dy, dz = corner & 1, (corner >> 1) & 1, (corner >> 2) & 1
                if dense:
                    idx = bidx + (dx + dy * r1 + dz * r1 * r1)
                else:
                    h = ((hx1 if dx else hx0) ^ (hy1 if dy else hy0)
                         ^ (hz1 if dz else hz0))
                    idx = ((h & msk) + jnp.uint32(off)).astype(jnp.int32)
                w = wxy[dy * 2 + dx] * (fz if dz else wz0)
                e = idx
                if cached:
                    idxc_v[corner, pl.ds(o, _L)] = e
                    wc_v[corner, pl.ds(o, _L)] = w
                else:
                    idx_b[buf][corner][pl.ds(o, _L)] = e
                    w_v[buf, corner, pl.ds(o, _L)] = w
            return cc

        lax.fori_loop(0, _G, grp_a, 0)

    def fire(buf):
        cps = []
        for c in range(8):
            cps.append(pltpu.async_copy(emb0_hbm.at[idx_b[buf][c]],
                                        rows_b[buf][2 * c], sems[buf]))
            cps.append(pltpu.async_copy(emb1_hbm.at[idx_b[buf][c]],
                                        rows_b[buf][2 * c + 1], sems[buf]))
        return cps

    def phase_c_dma(l, buf):
        def grp_c(g, cc):
            o = g * _L
            acc0 = jnp.zeros((_L,), jnp.float32)
            acc1 = jnp.zeros((_L,), jnp.float32)
            for corner in range(8):
                w = w_v[buf, corner, pl.ds(o, _L)]
                v0 = rows_b[buf][2 * corner][pl.ds(o, _L)]
                v1 = rows_b[buf][2 * corner + 1][pl.ds(o, _L)]
                acc0 = acc0 + w * v0
                acc1 = acc1 + w * v1
            r = l // 4
            fi = (2 * l) & 7
            ovec = (((r * (_C // 128) + g // 8) * 8 + fi) * 128
                    + (o & 127)) + iota
            plsc.store_scatter(out_v, [ovec], acc0)
            plsc.store_scatter(out_v, [ovec + 128], acc1)
            return cc

        lax.fori_loop(0, _G, grp_c, 0)

    def phase_c_cached(l):
        def grp_c(g, cc):
            o = g * _L
            acc0 = jnp.zeros((_L,), jnp.float32)
            acc1 = jnp.zeros((_L,), jnp.float32)
            for corner in range(8):
                w = wc_v[corner, pl.ds(o, _L)]
                evec = idxc_v[corner, pl.ds(o, _L)]
                v0 = plsc.load_gather(tab_v, [evec])
                v1 = plsc.load_gather(tab_v, [evec + _TAB_PAD])
                acc0 = acc0 + w * v0
                acc1 = acc1 + w * v1
            r = l // 4
            fi = (2 * l) & 7
            ovec = (((r * (_C // 128) + g // 8) * 8 + fi) * 128
                    + (o & 127)) + iota
            plsc.store_scatter(out_v, [ovec], acc0)
            plsc.store_scatter(out_v, [ovec + 128], acc1)
            return cc

        lax.fori_loop(0, _G, grp_c, 0)

    def chunk_body(ci, carry):
        cbase = base + ci * _C
        for a in range(3):
            pltpu.sync_copy(xt_hbm.at[pl.ds(a * _B + cbase, _C)],
                            xyz_v.at[pl.ds(a * _C, _C)])
        pend = None
        ndma = 0
        order = [2, 0, 3, 1] + list(range(4, _N_LEVELS))
        for l in order:
            cached = l < _N_CACHED
            buf = ndma % 2
            phase_a(l, buf, cached)
            if cached:
                phase_c_cached(l)
            else:
                copies = fire(buf)
                ndma += 1
                if pend is not None:
                    pl_, pb_, pc_ = pend
                    for cp in pc_:
                        cp.wait()
                    phase_c_dma(pl_, pb_)
                pend = (l, buf, copies)
        pl_, pb_, pc_ = pend
        for cp in pc_:
            cp.wait()
        phase_c_dma(pl_, pb_)
        for r in range(_OUT_RB):
            pltpu.sync_copy(
                out_v.at[pl.ds(r * (_C // 128) * 1024, (_C // 128) * 1024)],
                out_hbm.at[pl.ds((r * _PB + cbase // 128) * 1024,
                                 (_C // 128) * 1024)])
        return carry

    lax.fori_loop(0, _CHUNKS, chunk_body, 0)


def kernel(xyz, embeddings, min_xyz, max_xyz):
    xt = jnp.transpose(xyz).reshape(-1)
    emb0 = embeddings[:, 0]
    emb1 = embeddings[:, 1]
    inv = 1.0 / (max_xyz - min_xyz)
    mn3 = jnp.broadcast_to(min_xyz[:, None], (3, _L))
    inv3 = jnp.broadcast_to(inv[:, None], (3, _L))
    flat = _encode_sc(xt, emb0, emb1, mn3, inv3)
    return (flat.reshape(_OUT_RB, _PB, 8, 128)
            .transpose(1, 3, 0, 2).reshape(_B, _OUT_D))


# speedup vs baseline: 10.7879x; 1.8140x over previous
import functools

import jax
import jax.numpy as jnp
from jax import lax
from jax.experimental import pallas as pl
from jax.experimental.pallas import tpu as pltpu
from jax.experimental.pallas import tpu_sc as plsc

_N_LEVELS = 16
_N_FEAT = 2
_OFFSETS = [0, 4913, 14174, 31750, 67687, 136608, 269259, 543884, 1068172,
            1592460, 2116748, 2641036, 3165324, 3689612, 4213900, 4738188,
            5262476]
_RES = [16, 20, 25, 32, 40, 50, 64, 80, 101, 128, 161, 203, 256, 322, 406, 512]
_P2 = 2654435761
_P3 = 805459861

_B = 131072
_NC, _NS, _L = 2, 16, 16
_NW = _NC * _NS
_BPW = _B // _NW
_C = 512
_CHUNKS = _BPW // _C
_G = _C // _L
_OUT_D = _N_LEVELS * _N_FEAT

_N_CACHED = 4
_TAB_ROWS = _OFFSETS[_N_CACHED]
_TAB_PAD = ((_TAB_ROWS + 7) // 8) * 8
_VP_BLOCKS = (_OFFSETS[-1] + 127) // 128
_VP = _VP_BLOCKS * 128
_OUT_RB = _OUT_D // 8
_PB = _B // 128

_mesh = plsc.VectorSubcoreMesh(core_axis_name="c", subcore_axis_name="s")


@functools.partial(
    pl.kernel,
    mesh=_mesh,
    out_type=jax.ShapeDtypeStruct((_B * _OUT_D,), jnp.float32),
    scratch_types=[
        pltpu.VMEM((3, _L), jnp.float32),
        pltpu.VMEM((3, _L), jnp.float32),
        pltpu.VMEM((3 * _C,), jnp.float32),
        pltpu.VMEM((_TAB_PAD,), jnp.int32),
        [[pltpu.VMEM((_C,), jnp.int32) for _ in range(8)]
         for _ in range(2)],
        pltpu.VMEM((8, _C), jnp.int32),
        pltpu.VMEM((2, 8, _C), jnp.float32),
        pltpu.VMEM((8, _C), jnp.float32),
        [[pltpu.VMEM((_C,), jnp.int32) for _ in range(8)]
         for _ in range(2)],
        pltpu.VMEM((_C * _OUT_D,), jnp.float32),
        pltpu.SemaphoreType.DMA,
        pltpu.SemaphoreType.DMA,
    ],
    compiler_params=pltpu.CompilerParams(needs_layout_passes=False),
)
def _encode_sc(xt_hbm, emb_hbm, mn_hbm, inv_hbm, out_hbm,
               mn_v, inv_v, xyz_v, tab_v, idx_b, idxc_v, w_v, wc_v, rows_b,
               out_v, sem0, sem1):
    wid = lax.axis_index("s") * _NC + lax.axis_index("c")
    base = wid * _BPW
    pltpu.sync_copy(mn_hbm, mn_v)
    pltpu.sync_copy(inv_hbm, inv_v)
    pltpu.sync_copy(emb_hbm.at[pl.ds(0, _TAB_PAD)], tab_v)
    iota = lax.iota(jnp.int32, _L)
    sems = (sem0, sem1)

    def phase_a(l, buf, cached):
        res = _RES[l]
        size = _OFFSETS[l + 1] - _OFFSETS[l]
        off = _OFFSETS[l]
        dense = (res + 1) ** 3 <= size
        r1 = res + 1

        def grp_a(g, cc):
            o = g * _L
            ps, fs = [], []
            for a in range(3):
                xa = xyz_v[pl.ds(a * _C + o, _L)]
                xn = jnp.clip((xa - mn_v[a]) * inv_v[a], 0.0, 1.0)
                scl = xn * jnp.float32(res)
                p = jnp.minimum(scl.astype(jnp.int32), res - 1)
                ps.append(p)
                fs.append(scl - p.astype(jnp.float32))
            px, py, pz = ps
            fx, fy, fz = fs
            wx0 = 1.0 - fx
            wy0 = 1.0 - fy
            wz0 = 1.0 - fz
            wxy = (wx0 * wy0, fx * wy0, wx0 * fy, fx * fy)
            if not dense:
                hx0 = px.astype(jnp.uint32)
                hx1 = hx0 + jnp.uint32(1)
                hy0 = py.astype(jnp.uint32) * jnp.uint32(_P2)
                hy1 = hy0 + jnp.uint32(_P2)
                hz0 = pz.astype(jnp.uint32) * jnp.uint32(_P3)
                hz1 = hz0 + jnp.uint32(_P3)
                msk = jnp.uint32(size - 1)
            else:
                bidx = px + py * r1 + pz * (r1 * r1) + off
            for corner in range(8):
                dx, dy, dz = corner & 1, (corner >> 1) & 1, (corner >> 2) & 1
                if dense:
                    idx = bidx + (dx + dy * r1 + dz * r1 * r1)
                else:
                    h = ((hx1 if dx else hx0) ^ (hy1 if dy else hy0)
                         ^ (hz1 if dz else hz0))
                    idx = ((h & msk) + jnp.uint32(off)).astype(jnp.int32)
                w = wxy[dy * 2 + dx] * (fz if dz else wz0)
                e = idx
                if cached:
                    idxc_v[corner, pl.ds(o, _L)] = e
                    wc_v[corner, pl.ds(o, _L)] = w
                else:
                    idx_b[buf][corner][pl.ds(o, _L)] = e
                    w_v[buf, corner, pl.ds(o, _L)] = w
            return cc

        lax.fori_loop(0, _G, grp_a, 0)

    def fire(buf):
        return [pltpu.async_copy(emb_hbm.at[idx_b[buf][c]],
                                 rows_b[buf][c], sems[buf])
                for c in range(8)]

    def unpack2(vi):
        v0 = plsc.bitcast(vi << 16, jnp.float32)
        v1 = plsc.bitcast(vi & jnp.int32(-65536), jnp.float32)
        return v0, v1

    def phase_c_dma(l, buf):
        def grp_c(g, cc):
            o = g * _L
            acc0 = jnp.zeros((_L,), jnp.float32)
            acc1 = jnp.zeros((_L,), jnp.float32)
            for corner in range(8):
                w = w_v[buf, corner, pl.ds(o, _L)]
                v0, v1 = unpack2(rows_b[buf][corner][pl.ds(o, _L)])
                acc0 = acc0 + w * v0
                acc1 = acc1 + w * v1
            r = l // 4
            fi = (2 * l) & 7
            ovec = (((r * (_C // 128) + g // 8) * 8 + fi) * 128
                    + (o & 127)) + iota
            plsc.store_scatter(out_v, [ovec], acc0)
            plsc.store_scatter(out_v, [ovec + 128], acc1)
            return cc

        lax.fori_loop(0, _G, grp_c, 0)

    def phase_c_cached(l):
        def grp_c(g, cc):
            o = g * _L
            acc0 = jnp.zeros((_L,), jnp.float32)
            acc1 = jnp.zeros((_L,), jnp.float32)
            for corner in range(8):
                w = wc_v[corner, pl.ds(o, _L)]
                evec = idxc_v[corner, pl.ds(o, _L)]
                v0, v1 = unpack2(plsc.load_gather(tab_v, [evec]))
                acc0 = acc0 + w * v0
                acc1 = acc1 + w * v1
            r = l // 4
            fi = (2 * l) & 7
            ovec = (((r * (_C // 128) + g // 8) * 8 + fi) * 128
                    + (o & 127)) + iota
            plsc.store_scatter(out_v, [ovec], acc0)
            plsc.store_scatter(out_v, [ovec + 128], acc1)
            return cc

        lax.fori_loop(0, _G, grp_c, 0)

    def chunk_body(ci, carry):
        cbase = base + ci * _C
        for a in range(3):
            pltpu.sync_copy(xt_hbm.at[pl.ds(a * _B + cbase, _C)],
                            xyz_v.at[pl.ds(a * _C, _C)])
        pend = None
        ndma = 0
        order = [4, 0, 5, 1, 6, 2, 7, 3] + list(range(8, _N_LEVELS))
        for l in order:
            cached = l < _N_CACHED
            buf = ndma % 2
            phase_a(l, buf, cached)
            if cached:
                phase_c_cached(l)
            else:
                copies = fire(buf)
                ndma += 1
                if pend is not None:
                    pl_, pb_, pc_ = pend
                    for cp in pc_:
                        cp.wait()
                    phase_c_dma(pl_, pb_)
                pend = (l, buf, copies)
        pl_, pb_, pc_ = pend
        for cp in pc_:
            cp.wait()
        phase_c_dma(pl_, pb_)
        for r in range(_OUT_RB):
            pltpu.sync_copy(
                out_v.at[pl.ds(r * (_C // 128) * 1024, (_C // 128) * 1024)],
                out_hbm.at[pl.ds((r * _PB + cbase // 128) * 1024,
                                 (_C // 128) * 1024)])
        return carry

    lax.fori_loop(0, _CHUNKS, chunk_body, 0)


def kernel(xyz, embeddings, min_xyz, max_xyz):
    xt = jnp.transpose(xyz).reshape(-1)
    u0 = jax.lax.bitcast_convert_type(
        embeddings[:, 0].astype(jnp.bfloat16), jnp.uint16).astype(jnp.uint32)
    u1 = jax.lax.bitcast_convert_type(
        embeddings[:, 1].astype(jnp.bfloat16), jnp.uint16).astype(jnp.uint32)
    embp = jax.lax.bitcast_convert_type(u0 | (u1 << 16), jnp.int32)
    inv = 1.0 / (max_xyz - min_xyz)
    mn3 = jnp.broadcast_to(min_xyz[:, None], (3, _L))
    inv3 = jnp.broadcast_to(inv[:, None], (3, _L))
    flat = _encode_sc(xt, embp, mn3, inv3)
    return (flat.reshape(_OUT_RB, _PB, 8, 128)
            .transpose(1, 3, 0, 2).reshape(_B, _OUT_D))


# speedup vs baseline: 10.7927x; 1.0004x over previous
import functools

import jax
import jax.numpy as jnp
from jax import lax
from jax.experimental import pallas as pl
from jax.experimental.pallas import tpu as pltpu
from jax.experimental.pallas import tpu_sc as plsc

_N_LEVELS = 16
_N_FEAT = 2
_OFFSETS = [0, 4913, 14174, 31750, 67687, 136608, 269259, 543884, 1068172,
            1592460, 2116748, 2641036, 3165324, 3689612, 4213900, 4738188,
            5262476]
_RES = [16, 20, 25, 32, 40, 50, 64, 80, 101, 128, 161, 203, 256, 322, 406, 512]
_P2 = 2654435761
_P3 = 805459861

_B = 131072
_NC, _NS, _L = 2, 16, 16
_NW = _NC * _NS
_BPW = _B // _NW
_C = 512
_CHUNKS = _BPW // _C
_G = _C // _L
_OUT_D = _N_LEVELS * _N_FEAT

_N_CACHED = 4
_TAB_ROWS = _OFFSETS[_N_CACHED]
_TAB_PAD = ((_TAB_ROWS + 7) // 8) * 8
_VP_BLOCKS = (_OFFSETS[-1] + 127) // 128
_VP = _VP_BLOCKS * 128
_OUT_RB = _OUT_D // 8
_PB = _B // 128

_mesh = plsc.VectorSubcoreMesh(core_axis_name="c", subcore_axis_name="s")


@functools.partial(
    pl.kernel,
    mesh=_mesh,
    out_type=jax.ShapeDtypeStruct((_B * _OUT_D,), jnp.float32),
    scratch_types=[
        pltpu.VMEM((3, _L), jnp.float32),
        pltpu.VMEM((3, _L), jnp.float32),
        pltpu.VMEM((3 * _C,), jnp.float32),
        pltpu.VMEM((3 * _C,), jnp.float32),
        pltpu.VMEM((_TAB_PAD,), jnp.int32),
        [[pltpu.VMEM((_C,), jnp.int32) for _ in range(8)]
         for _ in range(2)],
        pltpu.VMEM((2, 8, _C), jnp.float32),
        [[pltpu.VMEM((_C,), jnp.int32) for _ in range(8)]
         for _ in range(2)],
        pltpu.VMEM((_C * _OUT_D,), jnp.float32),
        pltpu.SemaphoreType.DMA,
        pltpu.SemaphoreType.DMA,
    ],
    compiler_params=pltpu.CompilerParams(needs_layout_passes=False),
)
def _encode_sc(xt_hbm, emb_hbm, mn_hbm, inv_hbm, out_hbm,
               mn_v, inv_v, xyz_v, xn_v, tab_v, idx_b, w_v, rows_b,
               out_v, sem0, sem1):
    wid = lax.axis_index("s") * _NC + lax.axis_index("c")
    base = wid * _BPW
    pltpu.sync_copy(mn_hbm, mn_v)
    pltpu.sync_copy(inv_hbm, inv_v)
    pltpu.sync_copy(emb_hbm.at[pl.ds(0, _TAB_PAD)], tab_v)
    iota = lax.iota(jnp.int32, _L)
    sems = (sem0, sem1)

    def norm_chunk():
        def grp_n(g, cc):
            o = g * _L
            for a in range(3):
                xa = xyz_v[pl.ds(a * _C + o, _L)]
                xn_v[pl.ds(a * _C + o, _L)] = jnp.clip(
                    (xa - mn_v[a]) * inv_v[a], 0.0, 1.0)
            return cc

        lax.fori_loop(0, _G, grp_n, 0)

    def corner_setup(g, res, size, off, dense):
        o = g * _L
        r1 = res + 1
        ps, fs = [], []
        for a in range(3):
            xn = xn_v[pl.ds(a * _C + o, _L)]
            scl = xn * jnp.float32(res)
            pos = jnp.minimum(scl.astype(jnp.int32), res - 1)
            ps.append(pos)
            fs.append(scl - pos.astype(jnp.float32))
        px, py, pz = ps
        fx, fy, fz = fs
        wx0 = 1.0 - fx
        wy0 = 1.0 - fy
        wz0 = 1.0 - fz
        wxy = (wx0 * wy0, fx * wy0, wx0 * fy, fx * fy)
        idxs, ws = [], []
        if dense:
            bidx = px + py * r1 + pz * (r1 * r1) + off
        else:
            hx0 = px.astype(jnp.uint32)
            hx1 = hx0 + jnp.uint32(1)
            hy0 = py.astype(jnp.uint32) * jnp.uint32(_P2)
            hy1 = hy0 + jnp.uint32(_P2)
            hz0 = pz.astype(jnp.uint32) * jnp.uint32(_P3)
            hz1 = hz0 + jnp.uint32(_P3)
            msk = jnp.uint32(size - 1)
        for corner in range(8):
            dx, dy, dz = corner & 1, (corner >> 1) & 1, (corner >> 2) & 1
            if dense:
                idx = bidx + (dx + dy * r1 + dz * r1 * r1)
            else:
                h = ((hx1 if dx else hx0) ^ (hy1 if dy else hy0)
                     ^ (hz1 if dz else hz0))
                idx = ((h & msk) + jnp.uint32(off)).astype(jnp.int32)
            idxs.append(idx)
            ws.append(wxy[dy * 2 + dx] * (fz if dz else wz0))
        return idxs, ws

    def phase_a(l, buf):
        res = _RES[l]
        size = _OFFSETS[l + 1] - _OFFSETS[l]
        off = _OFFSETS[l]
        dense = (res + 1) ** 3 <= size

        def grp_a(g, cc):
            o = g * _L
            idxs, ws = corner_setup(g, res, size, off, dense)
            for corner in range(8):
                idx_b[buf][corner][pl.ds(o, _L)] = idxs[corner]
                w_v[buf, corner, pl.ds(o, _L)] = ws[corner]
            return cc

        lax.fori_loop(0, _G, grp_a, 0)

    def fire(buf):
        return [pltpu.async_copy(emb_hbm.at[idx_b[buf][c]],
                                 rows_b[buf][c], sems[buf])
                for c in range(8)]

    def unpack2(vi):
        v0 = plsc.bitcast(vi << 16, jnp.float32)
        v1 = plsc.bitcast(vi & jnp.int32(-65536), jnp.float32)
        return v0, v1

    def phase_c_dma(l, buf):
        r = l // 4
        fi = (2 * l) & 7

        def grp_c(g, cc):
            o = g * _L
            acc0 = jnp.zeros((_L,), jnp.float32)
            acc1 = jnp.zeros((_L,), jnp.float32)
            for corner in range(8):
                w = w_v[buf, corner, pl.ds(o, _L)]
                v0, v1 = unpack2(rows_b[buf][corner][pl.ds(o, _L)])
                acc0 = acc0 + w * v0
                acc1 = acc1 + w * v1
            base = ((r * (_C // 128) + g // 8) * 8 + fi) * 128 + (o & 127)
            out_v[pl.ds(base, _L)] = acc0
            out_v[pl.ds(base + 128, _L)] = acc1
            return cc

        lax.fori_loop(0, _G, grp_c, 0)

    def phase_cached(l):
        res = _RES[l]
        size = _OFFSETS[l + 1] - _OFFSETS[l]
        off = _OFFSETS[l]
        dense = (res + 1) ** 3 <= size
        r = l // 4
        fi = (2 * l) & 7

        def grp(g, cc):
            o = g * _L
            idxs, ws = corner_setup(g, res, size, off, dense)
            acc0 = jnp.zeros((_L,), jnp.float32)
            acc1 = jnp.zeros((_L,), jnp.float32)
            for corner in range(8):
                v0, v1 = unpack2(plsc.load_gather(tab_v, [idxs[corner]]))
                acc0 = acc0 + ws[corner] * v0
                acc1 = acc1 + ws[corner] * v1
            base = ((r * (_C // 128) + g // 8) * 8 + fi) * 128 + (o & 127)
            out_v[pl.ds(base, _L)] = acc0
            out_v[pl.ds(base + 128, _L)] = acc1
            return cc

        lax.fori_loop(0, _G, grp, 0)

    def chunk_body(ci, carry):
        cbase = base + ci * _C
        for a in range(3):
            pltpu.sync_copy(xt_hbm.at[pl.ds(a * _B + cbase, _C)],
                            xyz_v.at[pl.ds(a * _C, _C)])
        norm_chunk()
        pend = None
        ndma = 0
        order = [4, 0, 5, 1, 6, 2, 7, 3] + list(range(8, _N_LEVELS))
        for l in order:
            cached = l < _N_CACHED
            buf = ndma % 2
            if cached:
                phase_cached(l)
            else:
                phase_a(l, buf)
                copies = fire(buf)
                ndma += 1
                if pend is not None:
                    pl_, pb_, pc_ = pend
                    for cp in pc_:
                        cp.wait()
                    phase_c_dma(pl_, pb_)
                pend = (l, buf, copies)
        pl_, pb_, pc_ = pend
        for cp in pc_:
            cp.wait()
        phase_c_dma(pl_, pb_)
        for r in range(_OUT_RB):
            pltpu.sync_copy(
                out_v.at[pl.ds(r * (_C // 128) * 1024, (_C // 128) * 1024)],
                out_hbm.at[pl.ds((r * _PB + cbase // 128) * 1024,
                                 (_C // 128) * 1024)])
        return carry

    lax.fori_loop(0, _CHUNKS, chunk_body, 0)


def kernel(xyz, embeddings, min_xyz, max_xyz):
    xt = jnp.transpose(xyz).reshape(-1)
    u0 = jax.lax.bitcast_convert_type(
        embeddings[:, 0].astype(jnp.bfloat16), jnp.uint16).astype(jnp.uint32)
    u1 = jax.lax.bitcast_convert_type(
        embeddings[:, 1].astype(jnp.bfloat16), jnp.uint16).astype(jnp.uint32)
    embp = jax.lax.bitcast_convert_type(u0 | (u1 << 16), jnp.int32)
    inv = 1.0 / (max_xyz - min_xyz)
    mn3 = jnp.broadcast_to(min_xyz[:, None], (3, _L))
    inv3 = jnp.broadcast_to(inv[:, None], (3, _L))
    flat = _encode_sc(xt, embp, mn3, inv3)
    return (flat.reshape(_OUT_RB, _PB, 8, 128)
            .transpose(1, 3, 0, 2).reshape(_B, _OUT_D))


# speedup vs baseline: 11.3871x; 1.0551x over previous
import functools

import jax
import jax.numpy as jnp
from jax import lax
from jax.experimental import pallas as pl
from jax.experimental.pallas import tpu as pltpu
from jax.experimental.pallas import tpu_sc as plsc

_N_LEVELS = 16
_N_FEAT = 2
_OFFSETS = [0, 4913, 14174, 31750, 67687, 136608, 269259, 543884, 1068172,
            1592460, 2116748, 2641036, 3165324, 3689612, 4213900, 4738188,
            5262476]
_RES = [16, 20, 25, 32, 40, 50, 64, 80, 101, 128, 161, 203, 256, 322, 406, 512]
_P2 = 2654435761
_P3 = 805459861

_B = 131072
_NC, _NS, _L = 2, 16, 16
_NW = _NC * _NS
_BPW = _B // _NW
_C = 512
_CHUNKS = _BPW // _C
_G = _C // _L
_OUT_D = _N_LEVELS * _N_FEAT

_N_CACHED = 4
_TAB_ROWS = _OFFSETS[_N_CACHED]
_TAB_PAD = ((_TAB_ROWS + 7) // 8) * 8
_N_SPMEM = 6
_S_BASE = _OFFSETS[_N_CACHED]
_S_ROWS = _OFFSETS[_N_SPMEM] - _S_BASE
_S_PAD = ((_S_ROWS + 7) // 8) * 8
_VP_BLOCKS = (_OFFSETS[-1] + 127) // 128
_VP = _VP_BLOCKS * 128
_OUT_RB = _OUT_D // 8
_PB = _B // 128

_mesh = plsc.VectorSubcoreMesh(core_axis_name="c", subcore_axis_name="s")


@functools.partial(
    pl.kernel,
    mesh=_mesh,
    out_type=jax.ShapeDtypeStruct((_B * _OUT_D,), jnp.float32),
    scratch_types=[
        pltpu.VMEM((3, _L), jnp.float32),
        pltpu.VMEM((3, _L), jnp.float32),
        pltpu.VMEM((3 * _C,), jnp.float32),
        pltpu.VMEM((3 * _C,), jnp.float32),
        pltpu.VMEM((_TAB_PAD,), jnp.int32),
        pltpu.VMEM_SHARED((_S_PAD,), jnp.int32),
        [[pltpu.VMEM((_C,), jnp.int32) for _ in range(8)]
         for _ in range(2)],
        pltpu.VMEM((2, 8, _C), jnp.float32),
        [[pltpu.VMEM((_C,), jnp.int32) for _ in range(8)]
         for _ in range(2)],
        pltpu.VMEM((_C * _OUT_D,), jnp.float32),
        pltpu.SemaphoreType.DMA,
        pltpu.SemaphoreType.DMA,
    ],
    compiler_params=pltpu.CompilerParams(needs_layout_passes=False),
)
def _encode_sc(xt_hbm, emb_hbm, embm_hbm, mn_hbm, inv_hbm, out_hbm,
               mn_v, inv_v, xyz_v, xn_v, tab_v, stab_v, idx_b, w_v, rows_b,
               out_v, sem0, sem1):
    wid = lax.axis_index("s") * _NC + lax.axis_index("c")
    base = wid * _BPW
    pltpu.sync_copy(mn_hbm, mn_v)
    pltpu.sync_copy(inv_hbm, inv_v)
    pltpu.sync_copy(emb_hbm.at[pl.ds(0, _TAB_PAD)], tab_v)
    @pl.when(lax.axis_index("s") == 0)
    def _():
        pltpu.sync_copy(embm_hbm, stab_v)
    plsc.subcore_barrier()
    iota = lax.iota(jnp.int32, _L)
    sems = (sem0, sem1)

    def norm_chunk():
        def grp_n(g, cc):
            o = g * _L
            for a in range(3):
                xa = xyz_v[pl.ds(a * _C + o, _L)]
                xn_v[pl.ds(a * _C + o, _L)] = jnp.clip(
                    (xa - mn_v[a]) * inv_v[a], 0.0, 1.0)
            return cc

        lax.fori_loop(0, _G, grp_n, 0)

    def corner_setup(g, res, size, off, dense):
        o = g * _L
        r1 = res + 1
        ps, fs = [], []
        for a in range(3):
            xn = xn_v[pl.ds(a * _C + o, _L)]
            scl = xn * jnp.float32(res)
            pos = jnp.minimum(scl.astype(jnp.int32), res - 1)
            ps.append(pos)
            fs.append(scl - pos.astype(jnp.float32))
        px, py, pz = ps
        fx, fy, fz = fs
        wx0 = 1.0 - fx
        wy0 = 1.0 - fy
        wz0 = 1.0 - fz
        wxy = (wx0 * wy0, fx * wy0, wx0 * fy, fx * fy)
        idxs, ws = [], []
        if dense:
            bidx = px + py * r1 + pz * (r1 * r1) + off
        else:
            hx0 = px.astype(jnp.uint32)
            hx1 = hx0 + jnp.uint32(1)
            hy0 = py.astype(jnp.uint32) * jnp.uint32(_P2)
            hy1 = hy0 + jnp.uint32(_P2)
            hz0 = pz.astype(jnp.uint32) * jnp.uint32(_P3)
            hz1 = hz0 + jnp.uint32(_P3)
            msk = jnp.uint32(size - 1)
        for corner in range(8):
            dx, dy, dz = corner & 1, (corner >> 1) & 1, (corner >> 2) & 1
            if dense:
                idx = bidx + (dx + dy * r1 + dz * r1 * r1)
            else:
                h = ((hx1 if dx else hx0) ^ (hy1 if dy else hy0)
                     ^ (hz1 if dz else hz0))
                idx = ((h & msk) + jnp.uint32(off)).astype(jnp.int32)
            idxs.append(idx)
            ws.append(wxy[dy * 2 + dx] * (fz if dz else wz0))
        return idxs, ws

    def phase_a(l, buf):
        res = _RES[l]
        size = _OFFSETS[l + 1] - _OFFSETS[l]
        off = _OFFSETS[l] - (_S_BASE if l < _N_SPMEM else 0)
        dense = (res + 1) ** 3 <= size

        def grp_a(g, cc):
            o = g * _L
            idxs, ws = corner_setup(g, res, size, off, dense)
            for corner in range(8):
                idx_b[buf][corner][pl.ds(o, _L)] = idxs[corner]
                w_v[buf, corner, pl.ds(o, _L)] = ws[corner]
            return cc

        lax.fori_loop(0, _G, grp_a, 0)

    def fire(buf, spmem):
        src = stab_v if spmem else emb_hbm
        return [pltpu.async_copy(src.at[idx_b[buf][c]],
                                 rows_b[buf][c], sems[buf])
                for c in range(8)]

    def unpack2(vi):
        v0 = plsc.bitcast(vi << 16, jnp.float32)
        v1 = plsc.bitcast(vi & jnp.int32(-65536), jnp.float32)
        return v0, v1

    def phase_c_dma(l, buf):
        r = l // 4
        fi = (2 * l) & 7

        def grp_c(g, cc):
            o = g * _L
            acc0 = jnp.zeros((_L,), jnp.float32)
            acc1 = jnp.zeros((_L,), jnp.float32)
            for corner in range(8):
                w = w_v[buf, corner, pl.ds(o, _L)]
                v0, v1 = unpack2(rows_b[buf][corner][pl.ds(o, _L)])
                acc0 = acc0 + w * v0
                acc1 = acc1 + w * v1
            base = ((r * (_C // 128) + g // 8) * 8 + fi) * 128 + (o & 127)
            out_v[pl.ds(base, _L)] = acc0
            out_v[pl.ds(base + 128, _L)] = acc1
            return cc

        lax.fori_loop(0, _G, grp_c, 0)

    def phase_cached(l):
        res = _RES[l]
        size = _OFFSETS[l + 1] - _OFFSETS[l]
        off = _OFFSETS[l]
        dense = (res + 1) ** 3 <= size
        r = l // 4
        fi = (2 * l) & 7

        def grp(g, cc):
            o = g * _L
            idxs, ws = corner_setup(g, res, size, off, dense)
            acc0 = jnp.zeros((_L,), jnp.float32)
            acc1 = jnp.zeros((_L,), jnp.float32)
            for corner in range(8):
                v0, v1 = unpack2(plsc.load_gather(tab_v, [idxs[corner]]))
                acc0 = acc0 + ws[corner] * v0
                acc1 = acc1 + ws[corner] * v1
            base = ((r * (_C // 128) + g // 8) * 8 + fi) * 128 + (o & 127)
            out_v[pl.ds(base, _L)] = acc0
            out_v[pl.ds(base + 128, _L)] = acc1
            return cc

        lax.fori_loop(0, _G, grp, 0)

    def chunk_body(ci, carry):
        cbase = base + ci * _C
        for a in range(3):
            pltpu.sync_copy(xt_hbm.at[pl.ds(a * _B + cbase, _C)],
                            xyz_v.at[pl.ds(a * _C, _C)])
        norm_chunk()
        pend = None
        ndma = 0
        order = [4, 0, 5, 1, 6, 2, 7, 3] + list(range(8, _N_LEVELS))
        for l in order:
            cached = l < _N_CACHED
            buf = ndma % 2
            if cached:
                phase_cached(l)
            else:
                phase_a(l, buf)
                copies = fire(buf, l < _N_SPMEM)
                ndma += 1
                if pend is not None:
                    pl_, pb_, pc_ = pend
                    for cp in pc_:
                        cp.wait()
                    phase_c_dma(pl_, pb_)
                pend = (l, buf, copies)
        pl_, pb_, pc_ = pend
        for cp in pc_:
            cp.wait()
        phase_c_dma(pl_, pb_)
        for r in range(_OUT_RB):
            pltpu.sync_copy(
                out_v.at[pl.ds(r * (_C // 128) * 1024, (_C // 128) * 1024)],
                out_hbm.at[pl.ds((r * _PB + cbase // 128) * 1024,
                                 (_C // 128) * 1024)])
        return carry

    lax.fori_loop(0, _CHUNKS, chunk_body, 0)


def kernel(xyz, embeddings, min_xyz, max_xyz):
    xt = jnp.transpose(xyz).reshape(-1)
    u0 = jax.lax.bitcast_convert_type(
        embeddings[:, 0].astype(jnp.bfloat16), jnp.uint16).astype(jnp.uint32)
    u1 = jax.lax.bitcast_convert_type(
        embeddings[:, 1].astype(jnp.bfloat16), jnp.uint16).astype(jnp.uint32)
    embp = jax.lax.bitcast_convert_type(u0 | (u1 << 16), jnp.int32)
    inv = 1.0 / (max_xyz - min_xyz)
    mn3 = jnp.broadcast_to(min_xyz[:, None], (3, _L))
    inv3 = jnp.broadcast_to(inv[:, None], (3, _L))
    embm = jnp.pad(embp[_S_BASE:_OFFSETS[_N_SPMEM]],
                   (0, _S_PAD - _S_ROWS))
    flat = _encode_sc(xt, embp, embm, mn3, inv3)
    return (flat.reshape(_OUT_RB, _PB, 8, 128)
            .transpose(1, 3, 0, 2).reshape(_B, _OUT_D))


# speedup vs baseline: 11.6415x; 1.0223x over previous
import functools

import jax
import jax.numpy as jnp
from jax import lax
from jax.experimental import pallas as pl
from jax.experimental.pallas import tpu as pltpu
from jax.experimental.pallas import tpu_sc as plsc

_N_LEVELS = 16
_N_FEAT = 2
_OFFSETS = [0, 4913, 14174, 31750, 67687, 136608, 269259, 543884, 1068172,
            1592460, 2116748, 2641036, 3165324, 3689612, 4213900, 4738188,
            5262476]
_RES = [16, 20, 25, 32, 40, 50, 64, 80, 101, 128, 161, 203, 256, 322, 406, 512]
_P2 = 2654435761
_P3 = 805459861

_B = 131072
_NC, _NS, _L = 2, 16, 16
_NW = _NC * _NS
_BPW = _B // _NW
_C = 512
_CHUNKS = _BPW // _C
_G = _C // _L
_OUT_D = _N_LEVELS * _N_FEAT

_N_CACHED = 4
_TAB_ROWS = _OFFSETS[_N_CACHED]
_TAB_PAD = ((_TAB_ROWS + 7) // 8) * 8
_N_SPMEM = 6
_S_BASE = _OFFSETS[_N_CACHED]
_S_ROWS = _OFFSETS[_N_SPMEM] - _S_BASE
_S_PAD = ((_S_ROWS + 7) // 8) * 8
_VP_BLOCKS = (_OFFSETS[-1] + 127) // 128
_VP = _VP_BLOCKS * 128
_OUT_RB = _OUT_D // 8
_PB = _B // 128

_mesh = plsc.VectorSubcoreMesh(core_axis_name="c", subcore_axis_name="s")


@functools.partial(
    pl.kernel,
    mesh=_mesh,
    out_type=jax.ShapeDtypeStruct((_B * _OUT_D,), jnp.float32),
    scratch_types=[
        pltpu.VMEM((3, _L), jnp.float32),
        pltpu.VMEM((3, _L), jnp.float32),
        pltpu.VMEM((3 * _C,), jnp.float32),
        pltpu.VMEM((3 * _C,), jnp.float32),
        pltpu.VMEM((_TAB_PAD,), jnp.int32),
        pltpu.VMEM_SHARED((_S_PAD,), jnp.int32),
        [[pltpu.VMEM((_C,), jnp.int32) for _ in range(8)]
         for _ in range(2)],
        pltpu.VMEM((2, 8, _C), jnp.float32),
        [[pltpu.VMEM((_C,), jnp.int32) for _ in range(8)]
         for _ in range(2)],
        pltpu.VMEM((_C * _OUT_D,), jnp.float32),
        pltpu.SemaphoreType.DMA,
        pltpu.SemaphoreType.DMA,
    ],
    compiler_params=pltpu.CompilerParams(needs_layout_passes=False),
)
def _encode_sc(xt_hbm, emb_hbm, embm_hbm, mn_hbm, inv_hbm, out_hbm,
               mn_v, inv_v, xyz_v, xn_v, tab_v, stab_v, idx_b, w_v, rows_b,
               out_v, sem0, sem1):
    wid = lax.axis_index("s") * _NC + lax.axis_index("c")
    base = wid * _BPW
    pltpu.sync_copy(mn_hbm, mn_v)
    pltpu.sync_copy(inv_hbm, inv_v)
    pltpu.sync_copy(emb_hbm.at[pl.ds(0, _TAB_PAD)], tab_v)
    @pl.when(lax.axis_index("s") == 0)
    def _():
        pltpu.sync_copy(embm_hbm, stab_v)
    plsc.subcore_barrier()
    iota = lax.iota(jnp.int32, _L)
    sems = (sem0, sem1)

    def norm_chunk():
        def grp_n(g):
            o = g * _L
            for a in range(3):
                xa = xyz_v[pl.ds(a * _C + o, _L)]
                xn_v[pl.ds(a * _C + o, _L)] = jnp.clip(
                    (xa - mn_v[a]) * inv_v[a], 0.0, 1.0)

        plsc.parallel_loop(0, _G)(grp_n)

    def corner_setup(g, res, size, off, dense):
        o = g * _L
        r1 = res + 1
        ps, fs = [], []
        for a in range(3):
            xn = xn_v[pl.ds(a * _C + o, _L)]
            scl = xn * jnp.float32(res)
            pos = jnp.minimum(scl.astype(jnp.int32), res - 1)
            ps.append(pos)
            fs.append(scl - pos.astype(jnp.float32))
        px, py, pz = ps
        fx, fy, fz = fs
        wx0 = 1.0 - fx
        wy0 = 1.0 - fy
        wz0 = 1.0 - fz
        wxy = (wx0 * wy0, fx * wy0, wx0 * fy, fx * fy)
        idxs, ws = [], []
        if dense:
            bidx = px + py * r1 + pz * (r1 * r1) + off
        else:
            hx0 = px.astype(jnp.uint32)
            hx1 = hx0 + jnp.uint32(1)
            hy0 = py.astype(jnp.uint32) * jnp.uint32(_P2)
            hy1 = hy0 + jnp.uint32(_P2)
            hz0 = pz.astype(jnp.uint32) * jnp.uint32(_P3)
            hz1 = hz0 + jnp.uint32(_P3)
            msk = jnp.uint32(size - 1)
        for corner in range(8):
            dx, dy, dz = corner & 1, (corner >> 1) & 1, (corner >> 2) & 1
            if dense:
                idx = bidx + (dx + dy * r1 + dz * r1 * r1)
            else:
                h = ((hx1 if dx else hx0) ^ (hy1 if dy else hy0)
                     ^ (hz1 if dz else hz0))
                idx = ((h & msk) + jnp.uint32(off)).astype(jnp.int32)
            idxs.append(idx)
            ws.append(wxy[dy * 2 + dx] * (fz if dz else wz0))
        return idxs, ws

    def phase_a(l, buf):
        res = _RES[l]
        size = _OFFSETS[l + 1] - _OFFSETS[l]
        off = _OFFSETS[l] - (_S_BASE if l < _N_SPMEM else 0)
        dense = (res + 1) ** 3 <= size

        def grp_a(g):
            o = g * _L
            idxs, ws = corner_setup(g, res, size, off, dense)
            for corner in range(8):
                idx_b[buf][corner][pl.ds(o, _L)] = idxs[corner]
                w_v[buf, corner, pl.ds(o, _L)] = ws[corner]

        plsc.parallel_loop(0, _G)(grp_a)

    def fire(buf, spmem):
        src = stab_v if spmem else emb_hbm
        return [pltpu.async_copy(src.at[idx_b[buf][c]],
                                 rows_b[buf][c], sems[buf])
                for c in range(8)]

    def unpack2(vi):
        v0 = plsc.bitcast(vi << 16, jnp.float32)
        v1 = plsc.bitcast(vi & jnp.int32(-65536), jnp.float32)
        return v0, v1

    def phase_c_dma(l, buf):
        r = l // 4
        fi = (2 * l) & 7

        def grp_c(g):
            o = g * _L
            acc0 = jnp.zeros((_L,), jnp.float32)
            acc1 = jnp.zeros((_L,), jnp.float32)
            for corner in range(8):
                w = w_v[buf, corner, pl.ds(o, _L)]
                v0, v1 = unpack2(rows_b[buf][corner][pl.ds(o, _L)])
                acc0 = acc0 + w * v0
                acc1 = acc1 + w * v1
            base = ((r * (_C // 128) + g // 8) * 8 + fi) * 128 + (o & 127)
            out_v[pl.ds(base, _L)] = acc0
            out_v[pl.ds(base + 128, _L)] = acc1

        plsc.parallel_loop(0, _G)(grp_c)

    def phase_cached(l):
        res = _RES[l]
        size = _OFFSETS[l + 1] - _OFFSETS[l]
        off = _OFFSETS[l]
        dense = (res + 1) ** 3 <= size
        r = l // 4
        fi = (2 * l) & 7

        def grp(g):
            o = g * _L
            idxs, ws = corner_setup(g, res, size, off, dense)
            acc0 = jnp.zeros((_L,), jnp.float32)
            acc1 = jnp.zeros((_L,), jnp.float32)
            for corner in range(8):
                v0, v1 = unpack2(plsc.load_gather(tab_v, [idxs[corner]]))
                acc0 = acc0 + ws[corner] * v0
                acc1 = acc1 + ws[corner] * v1
            base = ((r * (_C // 128) + g // 8) * 8 + fi) * 128 + (o & 127)
            out_v[pl.ds(base, _L)] = acc0
            out_v[pl.ds(base + 128, _L)] = acc1

        plsc.parallel_loop(0, _G)(grp)

    def chunk_body(ci, carry):
        cbase = base + ci * _C
        for a in range(3):
            pltpu.sync_copy(xt_hbm.at[pl.ds(a * _B + cbase, _C)],
                            xyz_v.at[pl.ds(a * _C, _C)])
        norm_chunk()
        pend = None
        ndma = 0
        order = [4, 0, 5, 1, 6, 2, 7, 3] + list(range(8, _N_LEVELS))
        for l in order:
            cached = l < _N_CACHED
            buf = ndma % 2
            if cached:
                phase_cached(l)
            else:
                phase_a(l, buf)
                copies = fire(buf, l < _N_SPMEM)
                ndma += 1
                if pend is not None:
                    pl_, pb_, pc_ = pend
                    for cp in pc_:
                        cp.wait()
                    phase_c_dma(pl_, pb_)
                pend = (l, buf, copies)
        pl_, pb_, pc_ = pend
        for cp in pc_:
            cp.wait()
        phase_c_dma(pl_, pb_)
        for r in range(_OUT_RB):
            pltpu.sync_copy(
                out_v.at[pl.ds(r * (_C // 128) * 1024, (_C // 128) * 1024)],
                out_hbm.at[pl.ds((r * _PB + cbase // 128) * 1024,
                                 (_C // 128) * 1024)])
        return carry

    lax.fori_loop(0, _CHUNKS, chunk_body, 0)


def kernel(xyz, embeddings, min_xyz, max_xyz):
    xt = jnp.transpose(xyz).reshape(-1)
    u0 = jax.lax.bitcast_convert_type(
        embeddings[:, 0].astype(jnp.bfloat16), jnp.uint16).astype(jnp.uint32)
    u1 = jax.lax.bitcast_convert_type(
        embeddings[:, 1].astype(jnp.bfloat16), jnp.uint16).astype(jnp.uint32)
    embp = jax.lax.bitcast_convert_type(u0 | (u1 << 16), jnp.int32)
    inv = 1.0 / (max_xyz - min_xyz)
    mn3 = jnp.broadcast_to(min_xyz[:, None], (3, _L))
    inv3 = jnp.broadcast_to(inv[:, None], (3, _L))
    embm = jnp.pad(embp[_S_BASE:_OFFSETS[_N_SPMEM]],
                   (0, _S_PAD - _S_ROWS))
    flat = _encode_sc(xt, embp, embm, mn3, inv3)
    return (flat.reshape(_OUT_RB, _PB, 8, 128)
            .transpose(1, 3, 0, 2).reshape(_B, _OUT_D))


# speedup vs baseline: 11.6455x; 1.0003x over previous
import functools

import jax
import jax.numpy as jnp
from jax import lax
from jax.experimental import pallas as pl
from jax.experimental.pallas import tpu as pltpu
from jax.experimental.pallas import tpu_sc as plsc

_N_LEVELS = 16
_N_FEAT = 2
_OFFSETS = [0, 4913, 14174, 31750, 67687, 136608, 269259, 543884, 1068172,
            1592460, 2116748, 2641036, 3165324, 3689612, 4213900, 4738188,
            5262476]
_RES = [16, 20, 25, 32, 40, 50, 64, 80, 101, 128, 161, 203, 256, 322, 406, 512]
_P2 = 2654435761
_P3 = 805459861

_B = 131072
_NC, _NS, _L = 2, 16, 16
_NW = _NC * _NS
_BPW = _B // _NW
_C = 512
_CHUNKS = _BPW // _C
_G = _C // _L
_OUT_D = _N_LEVELS * _N_FEAT

_N_CACHED = 4
_TAB_ROWS = _OFFSETS[_N_CACHED]
_TAB_PAD = ((_TAB_ROWS + 7) // 8) * 8
_N_SPMEM = 6
_S_BASE = _OFFSETS[_N_CACHED]
_S_ROWS = _OFFSETS[_N_SPMEM] - _S_BASE
_S_PAD = ((_S_ROWS + 7) // 8) * 8
_VP_BLOCKS = (_OFFSETS[-1] + 127) // 128
_VP = _VP_BLOCKS * 128
_OUT_RB = _OUT_D // 8
_PB = _B // 128

_mesh = plsc.VectorSubcoreMesh(core_axis_name="c", subcore_axis_name="s")


@functools.partial(
    pl.kernel,
    mesh=_mesh,
    out_type=jax.ShapeDtypeStruct((_B * _OUT_D,), jnp.float32),
    scratch_types=[
        pltpu.VMEM((3, _L), jnp.float32),
        pltpu.VMEM((3, _L), jnp.float32),
        pltpu.VMEM((3 * _C,), jnp.float32),
        pltpu.VMEM((3 * _C,), jnp.float32),
        pltpu.VMEM((_TAB_PAD,), jnp.int32),
        pltpu.VMEM_SHARED((_S_PAD,), jnp.int32),
        [[pltpu.VMEM((_C,), jnp.int32) for _ in range(8)]
         for _ in range(2)],
        pltpu.VMEM((2, 8, _C), jnp.float32),
        [[pltpu.VMEM((_C,), jnp.int32) for _ in range(8)]
         for _ in range(2)],
        pltpu.VMEM((_C * _OUT_D,), jnp.float32),
        pltpu.SemaphoreType.DMA,
        pltpu.SemaphoreType.DMA,
    ],
    compiler_params=pltpu.CompilerParams(needs_layout_passes=False),
)
def _encode_sc(xt_hbm, emb_hbm, embm_hbm, mn_hbm, inv_hbm, out_hbm,
               mn_v, inv_v, xyz_v, xn_v, tab_v, stab_v, idx_b, w_v, rows_b,
               out_v, sem0, sem1):
    wid = lax.axis_index("s") * _NC + lax.axis_index("c")
    base = wid * _BPW
    pltpu.sync_copy(mn_hbm, mn_v)
    pltpu.sync_copy(inv_hbm, inv_v)
    pltpu.sync_copy(emb_hbm.at[pl.ds(0, _TAB_PAD)], tab_v)
    @pl.when(lax.axis_index("s") == 0)
    def _():
        pltpu.sync_copy(embm_hbm, stab_v)
    plsc.subcore_barrier()
    iota = lax.iota(jnp.int32, _L)
    sems = (sem0, sem1)

    def norm_chunk():
        def grp_n(g):
            o = g * _L
            for a in range(3):
                xa = xyz_v[pl.ds(a * _C + o, _L)]
                xn_v[pl.ds(a * _C + o, _L)] = jnp.clip(
                    (xa - mn_v[a]) * inv_v[a], 0.0, 1.0)

        plsc.parallel_loop(0, _G)(grp_n)

    def corner_setup(g, res, size, off, dense):
        o = g * _L
        r1 = res + 1
        ps, fs = [], []
        for a in range(3):
            xn = xn_v[pl.ds(a * _C + o, _L)]
            scl = xn * jnp.float32(res)
            pos = jnp.minimum(scl.astype(jnp.int32), res - 1)
            ps.append(pos)
            fs.append(scl - pos.astype(jnp.float32))
        px, py, pz = ps
        fx, fy, fz = fs
        wx0 = 1.0 - fx
        wy0 = 1.0 - fy
        wz0 = 1.0 - fz
        wxy = (wx0 * wy0, fx * wy0, wx0 * fy, fx * fy)
        idxs, ws = [], []
        if dense:
            bidx = px + py * r1 + pz * (r1 * r1) + off
        else:
            hx0 = px.astype(jnp.uint32)
            hx1 = hx0 + jnp.uint32(1)
            hy0 = py.astype(jnp.uint32) * jnp.uint32(_P2)
            hy1 = hy0 + jnp.uint32(_P2)
            hz0 = pz.astype(jnp.uint32) * jnp.uint32(_P3)
            hz1 = hz0 + jnp.uint32(_P3)
            msk = jnp.uint32(size - 1)
        for corner in range(8):
            dx, dy, dz = corner & 1, (corner >> 1) & 1, (corner >> 2) & 1
            if dense:
                idx = bidx + (dx + dy * r1 + dz * r1 * r1)
            else:
                h = ((hx1 if dx else hx0) ^ (hy1 if dy else hy0)
                     ^ (hz1 if dz else hz0))
                idx = ((h & msk) + jnp.uint32(off)).astype(jnp.int32)
            idxs.append(idx)
            ws.append(wxy[dy * 2 + dx] * (fz if dz else wz0))
        return idxs, ws

    def phase_a(l, buf):
        res = _RES[l]
        size = _OFFSETS[l + 1] - _OFFSETS[l]
        off = _OFFSETS[l] - (_S_BASE if l < _N_SPMEM else 0)
        dense = (res + 1) ** 3 <= size

        def grp_a(g):
            o = g * _L
            idxs, ws = corner_setup(g, res, size, off, dense)
            for corner in range(8):
                idx_b[buf][corner][pl.ds(o, _L)] = idxs[corner]
                w_v[buf, corner, pl.ds(o, _L)] = ws[corner]

        plsc.parallel_loop(0, _G, unroll=2)(grp_a)

    def fire(buf, spmem):
        src = stab_v if spmem else emb_hbm
        return [pltpu.async_copy(src.at[idx_b[buf][c]],
                                 rows_b[buf][c], sems[buf])
                for c in range(8)]

    def unpack2(vi):
        v0 = plsc.bitcast(vi << 16, jnp.float32)
        v1 = plsc.bitcast(vi & jnp.int32(-65536), jnp.float32)
        return v0, v1

    def phase_c_dma(l, buf):
        r = l // 4
        fi = (2 * l) & 7

        def grp_c(g):
            o = g * _L
            acc0 = jnp.zeros((_L,), jnp.float32)
            acc1 = jnp.zeros((_L,), jnp.float32)
            for corner in range(8):
                w = w_v[buf, corner, pl.ds(o, _L)]
                v0, v1 = unpack2(rows_b[buf][corner][pl.ds(o, _L)])
                acc0 = acc0 + w * v0
                acc1 = acc1 + w * v1
            base = ((r * (_C // 128) + g // 8) * 8 + fi) * 128 + (o & 127)
            out_v[pl.ds(base, _L)] = acc0
            out_v[pl.ds(base + 128, _L)] = acc1

        plsc.parallel_loop(0, _G, unroll=2)(grp_c)

    def phase_cached(l):
        res = _RES[l]
        size = _OFFSETS[l + 1] - _OFFSETS[l]
        off = _OFFSETS[l]
        dense = (res + 1) ** 3 <= size
        r = l // 4
        fi = (2 * l) & 7

        def grp(g):
            o = g * _L
            idxs, ws = corner_setup(g, res, size, off, dense)
            acc0 = jnp.zeros((_L,), jnp.float32)
            acc1 = jnp.zeros((_L,), jnp.float32)
            for corner in range(8):
                v0, v1 = unpack2(plsc.load_gather(tab_v, [idxs[corner]]))
                acc0 = acc0 + ws[corner] * v0
                acc1 = acc1 + ws[corner] * v1
            base = ((r * (_C // 128) + g // 8) * 8 + fi) * 128 + (o & 127)
            out_v[pl.ds(base, _L)] = acc0
            out_v[pl.ds(base + 128, _L)] = acc1

        plsc.parallel_loop(0, _G, unroll=2)(grp)

    def chunk_body(ci, carry):
        cbase = base + ci * _C
        for a in range(3):
            pltpu.sync_copy(xt_hbm.at[pl.ds(a * _B + cbase, _C)],
                            xyz_v.at[pl.ds(a * _C, _C)])
        norm_chunk()
        pend = None
        ndma = 0
        order = [4, 0, 5, 1, 6, 2, 7, 3] + list(range(8, _N_LEVELS))
        for l in order:
            cached = l < _N_CACHED
            buf = ndma % 2
            if cached:
                phase_cached(l)
            else:
                phase_a(l, buf)
                copies = fire(buf, l < _N_SPMEM)
                ndma += 1
                if pend is not None:
                    pl_, pb_, pc_ = pend
                    for cp in pc_:
                        cp.wait()
                    phase_c_dma(pl_, pb_)
                pend = (l, buf, copies)
        pl_, pb_, pc_ = pend
        for cp in pc_:
            cp.wait()
        phase_c_dma(pl_, pb_)
        for r in range(_OUT_RB):
            pltpu.sync_copy(
                out_v.at[pl.ds(r * (_C // 128) * 1024, (_C // 128) * 1024)],
                out_hbm.at[pl.ds((r * _PB + cbase // 128) * 1024,
                                 (_C // 128) * 1024)])
        return carry

    lax.fori_loop(0, _CHUNKS, chunk_body, 0)


def kernel(xyz, embeddings, min_xyz, max_xyz):
    xt = jnp.transpose(xyz).reshape(-1)
    u0 = jax.lax.bitcast_convert_type(
        embeddings[:, 0].astype(jnp.bfloat16), jnp.uint16).astype(jnp.uint32)
    u1 = jax.lax.bitcast_convert_type(
        embeddings[:, 1].astype(jnp.bfloat16), jnp.uint16).astype(jnp.uint32)
    embp = jax.lax.bitcast_convert_type(u0 | (u1 << 16), jnp.int32)
    inv = 1.0 / (max_xyz - min_xyz)
    mn3 = jnp.broadcast_to(min_xyz[:, None], (3, _L))
    inv3 = jnp.broadcast_to(inv[:, None], (3, _L))
    embm = jnp.pad(embp[_S_BASE:_OFFSETS[_N_SPMEM]],
                   (0, _S_PAD - _S_ROWS))
    flat = _encode_sc(xt, embp, embm, mn3, inv3)
    return (flat.reshape(_OUT_RB, _PB, 8, 128)
            .transpose(1, 3, 0, 2).reshape(_B, _OUT_D))
